# baseline (speedup 1.0000x reference)
import functools

import jax
import jax.numpy as jnp
from jax import lax
from jax.experimental import pallas as pl
from jax.experimental.pallas import tpu as pltpu
from jax.experimental.pallas import tpu_sc as plsc

_N = 10000
_NPAD = 10240
_E = 320000
_PD = 128
_B = 16
_NC = 2
_NS = 16
_NT = _NC * _NS
_K = 80
_NCH = 128
_EPT = _NCH * _K
_EPAD = _NT * _EPT
_RPT = _NPAD // _NS

_SLOPE = 0.01


def _leaky(x):
    return jnp.where(x >= 0, x, _SLOPE * x)


def _sc_segsum(table, src3, dst3, zeros):
    mesh = plsc.VectorSubcoreMesh(core_axis_name="c", subcore_axis_name="s")

    @functools.partial(
        pl.kernel,
        mesh=mesh,
        out_type=jax.ShapeDtypeStruct((_NC * _NPAD, _PD), jnp.float32),
        scratch_types=[
            pltpu.VMEM((_K,), jnp.int32),
            pltpu.VMEM((_K,), jnp.int32),
            pltpu.VMEM((_K, _PD), jnp.float32),
            pltpu.VMEM_SHARED((_NPAD, _PD), jnp.float32),
        ],
    )
    def k(table_h, src_h, dst_h, zeros_h, out_h,
          src_v, dst_v, rows, acc_sh):
        cid = lax.axis_index("c")
        sid = lax.axis_index("s")
        wid = cid * _NS + sid
        pltpu.sync_copy(zeros_h, acc_sh.at[pl.ds(sid * _RPT, _RPT)])
        plsc.subcore_barrier()

        def body(c, carry):
            pltpu.sync_copy(src_h.at[pl.ds(wid * _EPT + c * _K, _K)], src_v)
            pltpu.sync_copy(dst_h.at[wid, c], dst_v)
            pltpu.sync_copy(table_h.at[src_v], rows)
            pltpu.sync_copy(rows, acc_sh.at[dst_v], add=True)
            return carry

        lax.fori_loop(0, _NCH, body, 0)
        plsc.subcore_barrier()
        pltpu.sync_copy(
            acc_sh.at[pl.ds(sid * _RPT, _RPT)],
            out_h.at[pl.ds(cid * _NPAD + sid * _RPT, _RPT)])

    return k(table, src3, dst3, zeros)


_EB = 4000


def _ti_body(a_ref, w_ref, b_ref, o_ref):
    x = a_ref[...] * w_ref[...] + b_ref[...]
    o_ref[...] = _leaky(x)


def _tc_ti(edge_attr, W3, b3):
    return pl.pallas_call(
        _ti_body,
        grid=(_E // _EB,),
        in_specs=[
            pl.BlockSpec((_EB, 1), lambda i: (i, 0)),
            pl.BlockSpec((1, _PD), lambda i: (0, 0)),
            pl.BlockSpec((1, _PD), lambda i: (0, 0)),
        ],
        out_specs=pl.BlockSpec((_EB, _PD), lambda i: (i, 0)),
        out_shape=jax.ShapeDtypeStruct((_E, _PD), jnp.float32),
    )(edge_attr, W3, b3.reshape(1, _PD))


_RB = 512


def _c_body(t0_ref, t1_ref, af_ref, w2_ref, wb_ref, b1_ref, b2_ref, o_ref):
    t = t0_ref[...] + t1_ref[...]
    o_ref[...] = (
        jnp.dot(t, w2_ref[...], preferred_element_type=jnp.float32)
        + jnp.dot(af_ref[...], wb_ref[...], preferred_element_type=jnp.float32)
        + b1_ref[...] + b2_ref[...])


def _tc_c(aggti, af, W2, wb, b1, b2):
    nb = _NPAD // _RB
    return pl.pallas_call(
        _c_body,
        grid=(nb,),
        in_specs=[
            pl.BlockSpec((_RB, _PD), lambda i: (i, 0)),
            pl.BlockSpec((_RB, _PD), lambda i, nb=nb: (i + nb, 0)),
            pl.BlockSpec((_RB, 16), lambda i: (i, 0)),
            pl.BlockSpec((_PD, _PD), lambda i: (0, 0)),
            pl.BlockSpec((16, _PD), lambda i: (0, 0)),
            pl.BlockSpec((1, _PD), lambda i: (0, 0)),
            pl.BlockSpec((1, _PD), lambda i: (0, 0)),
        ],
        out_specs=pl.BlockSpec((_RB, _PD), lambda i: (i, 0)),
        out_shape=jax.ShapeDtypeStruct((_NPAD, _PD), jnp.float32),
    )(aggti, aggti, af, W2, wb, b1.reshape(1, _PD), b2.reshape(1, _PD))


def _round_body(a0_ref, a1_ref, w1_ref, c_ref, o_ref):
    t = a0_ref[...] + a1_ref[...]
    x = jnp.dot(t, w1_ref[...], preferred_element_type=jnp.float32) + c_ref[...]
    o_ref[...] = _leaky(x)


def _tc_round(agg, W1, C):
    nb = _NPAD // _RB
    return pl.pallas_call(
        _round_body,
        grid=(nb,),
        in_specs=[
            pl.BlockSpec((_RB, _PD), lambda i: (i, 0)),
            pl.BlockSpec((_RB, _PD), lambda i, nb=nb: (i + nb, 0)),
            pl.BlockSpec((_PD, _PD), lambda i: (0, 0)),
            pl.BlockSpec((_RB, _PD), lambda i: (i, 0)),
        ],
        out_specs=pl.BlockSpec((_RB, _PD), lambda i: (i, 0)),
        out_shape=jax.ShapeDtypeStruct((_NPAD, _PD), jnp.float32),
    )(agg, agg, W1, C)


def _final_body(mu_ref, bf_ref, wc_ref, bc_ref, o_ref, s_sum, s_cnt):
    i = pl.program_id(0)

    @pl.when(i == 0)
    def _():
        s_sum[...] = jnp.zeros_like(s_sum)
        s_cnt[...] = jnp.zeros_like(s_cnt)

    iot = lax.broadcasted_iota(jnp.int32, (1, _B), 1).astype(jnp.float32)
    onehot = (bf_ref[...] == iot)
    onehot = onehot.astype(jnp.float32)
    dn = (((0,), (0,)), ((), ()))
    s_sum[...] += lax.dot_general(onehot, mu_ref[...], dn,
                                  preferred_element_type=jnp.float32)
    s_cnt[...] += lax.dot_general(onehot, jnp.ones((_RB, 1), jnp.float32), dn,
                                  preferred_element_type=jnp.float32)

    @pl.when(i == pl.num_programs(0) - 1)
    def _():
        g = s_sum[...] / jnp.maximum(s_cnt[...], 1.0)
        z = jnp.dot(g, wc_ref[...], preferred_element_type=jnp.float32) + bc_ref[...]
        o_ref[...] = 1.0 / (1.0 + jnp.exp(-z))


def _tc_final(mu, bf, Wc, bc):
    nb = _NPAD // _RB
    return pl.pallas_call(
        _final_body,
        grid=(nb,),
        in_specs=[
            pl.BlockSpec((_RB, _PD), lambda i: (i, 0)),
            pl.BlockSpec((_RB, 1), lambda i: (i, 0)),
            pl.BlockSpec((_PD, 1), lambda i: (0, 0)),
            pl.BlockSpec((1, 1), lambda i: (0, 0)),
        ],
        out_specs=pl.BlockSpec((_B, 1), lambda i: (0, 0)),
        out_shape=jax.ShapeDtypeStruct((_B, 1), jnp.float32),
        scratch_shapes=[
            pltpu.VMEM((_B, _PD), jnp.float32),
            pltpu.VMEM((_B, 1), jnp.float32),
        ],
    )(mu, bf, Wc, bc.reshape(1, 1))


def kernel(x_vehicle, x_pickup, x_dropoff, edge_attr, node_mu,
           W1, b1, W2, b2, W3, b3, Wv, bv, Wp, bp, Wd, bd, Wc, bc,
           edge_index, node_types, batch):
    V = x_vehicle.shape[0]
    P = x_pickup.shape[0]
    D = x_dropoff.shape[0]
    src = jnp.pad(edge_index[0], (0, _EPAD - _E))
    dst = jnp.pad(edge_index[1], (0, _EPAD - _E),
                  constant_values=_N).reshape(_NT, _NCH, _K)
    zeros = jnp.zeros((_RPT, _PD), jnp.float32)

    T = _tc_ti(edge_attr, W3, b3)
    iota = jnp.pad(jnp.arange(_E, dtype=jnp.int32), (0, _EPAD - _E))
    aggti = _sc_segsum(T, iota, dst, zeros)

    af = jnp.zeros((_NPAD, 16), jnp.float32)
    af = af.at[:V, 0:2].set(x_vehicle)
    af = af.at[V:V + P, 2:5].set(x_pickup)
    af = af.at[V + P:V + P + D, 5:7].set(x_dropoff)
    af = af.at[:V, 7].set(1.0)
    af = af.at[V:V + P, 8].set(1.0)
    af = af.at[V + P:V + P + D, 9].set(1.0)
    wb = jnp.concatenate(
        [Wv, Wp, Wd, bv[None], bp[None], bd[None],
         jnp.zeros((6, _PD), jnp.float32)], axis=0)

    C = _tc_c(aggti, af, W2, wb, b1, b2)

    mu = jnp.pad(node_mu, ((0, _NPAD - _N), (0, 0)))
    for _ in range(4):
        agg = _sc_segsum(mu, src, dst, zeros)
        mu = _tc_round(agg, W1, C)

    bf = jnp.pad(batch.astype(jnp.float32), (0, _NPAD - _N),
                 constant_values=float(_B)).reshape(_NPAD, 1)
    return _tc_final(mu, bf, Wc, bc)


# speedup vs baseline: 1.3035x; 1.3035x over previous
import functools

import jax
import jax.numpy as jnp
from jax import lax
from jax.experimental import pallas as pl
from jax.experimental.pallas import tpu as pltpu
from jax.experimental.pallas import tpu_sc as plsc

_N = 10000
_NPAD = 10240
_E = 320000
_PD = 128
_B = 16
_NC = 2
_NS = 16
_NT = _NC * _NS
_K = 80
_NCH = 128
_EPT = _NCH * _K
_EPAD = _NT * _EPT
_RPT = _NPAD // _NS

_SLOPE = 0.01


def _leaky(x):
    return jnp.where(x >= 0, x, _SLOPE * x)


def _sc_segsum(table, src3, dst3, zeros):
    mesh = plsc.VectorSubcoreMesh(core_axis_name="c", subcore_axis_name="s")

    @functools.partial(
        pl.kernel,
        mesh=mesh,
        out_type=jax.ShapeDtypeStruct((_NC * _NPAD, _PD), jnp.float32),
        scratch_types=[
            pltpu.VMEM((_EPT,), jnp.int32),
            pltpu.VMEM((_NCH, _K), jnp.int32),
            pltpu.VMEM((_K, _PD), jnp.float32),
            pltpu.VMEM((_K, _PD), jnp.float32),
            pltpu.VMEM_SHARED((_NPAD, _PD), jnp.float32),
            pltpu.SemaphoreType.DMA,
            pltpu.SemaphoreType.DMA,
            pltpu.SemaphoreType.DMA,
            pltpu.SemaphoreType.DMA,
        ],
    )
    def k(table_h, src_h, dst_h, zeros_h, out_h,
          src_v, dst_v, rows0, rows1, acc_sh, sg0, sg1, ss0, ss1):
        cid = lax.axis_index("c")
        sid = lax.axis_index("s")
        wid = cid * _NS + sid
        pltpu.sync_copy(src_h.at[pl.ds(wid * _EPT, _EPT)], src_v)
        pltpu.sync_copy(dst_h.at[wid], dst_v)

        def _gather(c, buf, sem):
            pltpu.async_copy(table_h.at[src_v.at[pl.ds(c * _K, _K)]], buf, sem)

        def _gather_wait(c, buf, sem):
            pltpu.make_async_copy(
                table_h.at[src_v.at[pl.ds(c * _K, _K)]], buf, sem).wait()

        def _scat(c, buf, sem):
            pltpu.async_copy(buf, acc_sh.at[dst_v.at[c]], sem, add=True)

        def _scat_wait(c, buf, sem):
            pltpu.make_async_copy(buf, acc_sh.at[dst_v.at[c]], sem).wait()

        _gather(0, rows0, sg0)
        pltpu.sync_copy(zeros_h, acc_sh.at[pl.ds(sid * _RPT, _RPT)])
        plsc.subcore_barrier()

        def body(p, carry):
            c0 = 2 * p

            @pl.when(p > 0)
            def _():
                _scat_wait(c0 - 1, rows1, ss1)

            _gather(c0 + 1, rows1, sg1)
            _gather_wait(c0, rows0, sg0)
            _scat(c0, rows0, ss0)
            _gather_wait(c0 + 1, rows1, sg1)
            _scat(c0 + 1, rows1, ss1)
            _scat_wait(c0, rows0, ss0)

            @pl.when(p + 1 < _NCH // 2)
            def _():
                _gather(c0 + 2, rows0, sg0)

            return carry

        lax.fori_loop(0, _NCH // 2, body, 0)
        _scat_wait(_NCH - 1, rows1, ss1)
        plsc.subcore_barrier()
        pltpu.sync_copy(
            acc_sh.at[pl.ds(sid * _RPT, _RPT)],
            out_h.at[pl.ds(cid * _NPAD + sid * _RPT, _RPT)])

    return k(table, src3, dst3, zeros)


_EB = 4000


def _ti_body(a_ref, w_ref, b_ref, o_ref):
    x = a_ref[...] * w_ref[...] + b_ref[...]
    o_ref[...] = _leaky(x)


def _tc_ti(edge_attr, W3, b3):
    return pl.pallas_call(
        _ti_body,
        grid=(_E // _EB,),
        in_specs=[
            pl.BlockSpec((_EB, 1), lambda i: (i, 0)),
            pl.BlockSpec((1, _PD), lambda i: (0, 0)),
            pl.BlockSpec((1, _PD), lambda i: (0, 0)),
        ],
        out_specs=pl.BlockSpec((_EB, _PD), lambda i: (i, 0)),
        out_shape=jax.ShapeDtypeStruct((_E, _PD), jnp.float32),
    )(edge_attr, W3, b3.reshape(1, _PD))


_RB = 512


def _c_body(t0_ref, t1_ref, af_ref, w2_ref, wb_ref, b1_ref, b2_ref, o_ref):
    t = t0_ref[...] + t1_ref[...]
    o_ref[...] = (
        jnp.dot(t, w2_ref[...], preferred_element_type=jnp.float32)
        + jnp.dot(af_ref[...], wb_ref[...], preferred_element_type=jnp.float32)
        + b1_ref[...] + b2_ref[...])


def _tc_c(aggti, af, W2, wb, b1, b2):
    nb = _NPAD // _RB
    return pl.pallas_call(
        _c_body,
        grid=(nb,),
        in_specs=[
            pl.BlockSpec((_RB, _PD), lambda i: (i, 0)),
            pl.BlockSpec((_RB, _PD), lambda i, nb=nb: (i + nb, 0)),
            pl.BlockSpec((_RB, 16), lambda i: (i, 0)),
            pl.BlockSpec((_PD, _PD), lambda i: (0, 0)),
            pl.BlockSpec((16, _PD), lambda i: (0, 0)),
            pl.BlockSpec((1, _PD), lambda i: (0, 0)),
            pl.BlockSpec((1, _PD), lambda i: (0, 0)),
        ],
        out_specs=pl.BlockSpec((_RB, _PD), lambda i: (i, 0)),
        out_shape=jax.ShapeDtypeStruct((_NPAD, _PD), jnp.float32),
    )(aggti, aggti, af, W2, wb, b1.reshape(1, _PD), b2.reshape(1, _PD))


def _round_body(a0_ref, a1_ref, w1_ref, c_ref, o_ref):
    t = a0_ref[...] + a1_ref[...]
    x = jnp.dot(t, w1_ref[...], preferred_element_type=jnp.float32) + c_ref[...]
    o_ref[...] = _leaky(x)


def _tc_round(agg, W1, C):
    nb = _NPAD // _RB
    return pl.pallas_call(
        _round_body,
        grid=(nb,),
        in_specs=[
            pl.BlockSpec((_RB, _PD), lambda i: (i, 0)),
            pl.BlockSpec((_RB, _PD), lambda i, nb=nb: (i + nb, 0)),
            pl.BlockSpec((_PD, _PD), lambda i: (0, 0)),
            pl.BlockSpec((_RB, _PD), lambda i: (i, 0)),
        ],
        out_specs=pl.BlockSpec((_RB, _PD), lambda i: (i, 0)),
        out_shape=jax.ShapeDtypeStruct((_NPAD, _PD), jnp.float32),
    )(agg, agg, W1, C)


def _final_body(mu_ref, bf_ref, wc_ref, bc_ref, o_ref, s_sum, s_cnt):
    i = pl.program_id(0)

    @pl.when(i == 0)
    def _():
        s_sum[...] = jnp.zeros_like(s_sum)
        s_cnt[...] = jnp.zeros_like(s_cnt)

    iot = lax.broadcasted_iota(jnp.int32, (1, _B), 1).astype(jnp.float32)
    onehot = (bf_ref[...] == iot)
    onehot = onehot.astype(jnp.float32)
    dn = (((0,), (0,)), ((), ()))
    s_sum[...] += lax.dot_general(onehot, mu_ref[...], dn,
                                  preferred_element_type=jnp.float32)
    s_cnt[...] += lax.dot_general(onehot, jnp.ones((_RB, 1), jnp.float32), dn,
                                  preferred_element_type=jnp.float32)

    @pl.when(i == pl.num_programs(0) - 1)
    def _():
        g = s_sum[...] / jnp.maximum(s_cnt[...], 1.0)
        z = jnp.dot(g, wc_ref[...], preferred_element_type=jnp.float32) + bc_ref[...]
        o_ref[...] = 1.0 / (1.0 + jnp.exp(-z))


def _tc_final(mu, bf, Wc, bc):
    nb = _NPAD // _RB
    return pl.pallas_call(
        _final_body,
        grid=(nb,),
        in_specs=[
            pl.BlockSpec((_RB, _PD), lambda i: (i, 0)),
            pl.BlockSpec((_RB, 1), lambda i: (i, 0)),
            pl.BlockSpec((_PD, 1), lambda i: (0, 0)),
            pl.BlockSpec((1, 1), lambda i: (0, 0)),
        ],
        out_specs=pl.BlockSpec((_B, 1), lambda i: (0, 0)),
        out_shape=jax.ShapeDtypeStruct((_B, 1), jnp.float32),
        scratch_shapes=[
            pltpu.VMEM((_B, _PD), jnp.float32),
            pltpu.VMEM((_B, 1), jnp.float32),
        ],
    )(mu, bf, Wc, bc.reshape(1, 1))


def kernel(x_vehicle, x_pickup, x_dropoff, edge_attr, node_mu,
           W1, b1, W2, b2, W3, b3, Wv, bv, Wp, bp, Wd, bd, Wc, bc,
           edge_index, node_types, batch):
    V = x_vehicle.shape[0]
    P = x_pickup.shape[0]
    D = x_dropoff.shape[0]
    src = jnp.pad(edge_index[0], (0, _EPAD - _E))
    dst = jnp.pad(edge_index[1], (0, _EPAD - _E),
                  constant_values=_N).reshape(_NT, _NCH, _K)
    zeros = jnp.zeros((_RPT, _PD), jnp.float32)

    T = _tc_ti(edge_attr, W3, b3)
    iota = jnp.pad(jnp.arange(_E, dtype=jnp.int32), (0, _EPAD - _E))
    aggti = _sc_segsum(T, iota, dst, zeros)

    af = jnp.zeros((_NPAD, 16), jnp.float32)
    af = af.at[:V, 0:2].set(x_vehicle)
    af = af.at[V:V + P, 2:5].set(x_pickup)
    af = af.at[V + P:V + P + D, 5:7].set(x_dropoff)
    af = af.at[:V, 7].set(1.0)
    af = af.at[V:V + P, 8].set(1.0)
    af = af.at[V + P:V + P + D, 9].set(1.0)
    wb = jnp.concatenate(
        [Wv, Wp, Wd, bv[None], bp[None], bd[None],
         jnp.zeros((6, _PD), jnp.float32)], axis=0)

    C = _tc_c(aggti, af, W2, wb, b1, b2)

    mu = jnp.pad(node_mu, ((0, _NPAD - _N), (0, 0)))
    for _ in range(4):
        agg = _sc_segsum(mu, src, dst, zeros)
        mu = _tc_round(agg, W1, C)

    bf = jnp.pad(batch.astype(jnp.float32), (0, _NPAD - _N),
                 constant_values=float(_B)).reshape(_NPAD, 1)
    return _tc_final(mu, bf, Wc, bc)


# speedup vs baseline: 1.5193x; 1.1655x over previous
import functools

import jax
import jax.numpy as jnp
from jax import lax
from jax.experimental import pallas as pl
from jax.experimental.pallas import tpu as pltpu
from jax.experimental.pallas import tpu_sc as plsc

_N = 10000
_NPAD = 10240
_E = 320000
_PD = 128
_B = 16
_NC = 2
_NS = 16
_NT = _NC * _NS
_W = _NPAD // _NT
_K = 256
_ACC = _W + 8

_SLOPE = 0.01


def _leaky(x):
    return jnp.where(x >= 0, x, _SLOPE * x)


def _sc_segsum(table, gidx, dsts, bounds):
    mesh = plsc.VectorSubcoreMesh(core_axis_name="c", subcore_axis_name="s")

    @functools.partial(
        pl.kernel,
        mesh=mesh,
        out_type=jax.ShapeDtypeStruct((_NPAD, _PD), jnp.float32),
        scratch_types=[
            pltpu.VMEM((64,), jnp.int32),
            pltpu.VMEM((_K,), jnp.int32),
            pltpu.VMEM((_K,), jnp.int32),
            pltpu.VMEM((_K,), jnp.int32),
            pltpu.VMEM((_K,), jnp.int32),
            pltpu.VMEM((_K, _PD), jnp.float32),
            pltpu.VMEM((_K, _PD), jnp.float32),
            pltpu.VMEM((_ACC, _PD), jnp.float32),
            pltpu.SemaphoreType.DMA,
            pltpu.SemaphoreType.DMA,
            pltpu.SemaphoreType.DMA,
            pltpu.SemaphoreType.DMA,
        ],
    )
    def k(table_h, src_h, dst_h, bounds_h, out_h,
          p_v, s0, s1, d0, d1, rows0, rows1, acc, si0, si1, sg0, sg1):
        cid = lax.axis_index("c")
        sid = lax.axis_index("s")
        wid = cid * _NS + sid
        base = wid * _W
        pltpu.sync_copy(bounds_h, p_v)
        pv = p_v[pl.ds(wid, 16)]
        start = pl.multiple_of(lax.div(pv[0], 128) * 128, 128)
        cnt = pv[1] - start
        nch = lax.div(cnt + (_K - 1), _K)

        zv = jnp.zeros((16,), jnp.float32)

        def zbody(r, carry):
            for g in range(_PD // 16):
                acc[r, pl.ds(g * 16, 16)] = zv
            return carry

        lax.fori_loop(0, _ACC, zbody, 0)

        def _off(c):
            return pl.multiple_of(start + c * _K, 128)

        def _sd(c, sbuf, dbuf, sem):
            pltpu.async_copy(src_h.at[pl.ds(_off(c), _K)], sbuf, sem)
            pltpu.async_copy(dst_h.at[pl.ds(_off(c), _K)], dbuf, sem)

        def _sd_wait(c, sbuf, dbuf, sem):
            pltpu.make_async_copy(
                src_h.at[pl.ds(_off(c), _K)], sbuf, sem).wait()
            pltpu.make_async_copy(
                dst_h.at[pl.ds(_off(c), _K)], dbuf, sem).wait()

        def _gather(sbuf, buf, sem):
            pltpu.async_copy(table_h.at[sbuf], buf, sem)

        def _gather_wait(sbuf, buf, sem):
            pltpu.make_async_copy(table_h.at[sbuf], buf, sem).wait()

        def _compute(dbuf, rows):
            def gbody(q, carry):
                e0 = q * 16
                dv = dbuf[pl.ds(e0, 16)]
                lv = jnp.where((dv >= base) & (dv < base + _W), dv - base, _W)
                for i in range(16):
                    l = lv[i]
                    for g in range(_PD // 16):
                        x = rows[e0 + i, pl.ds(g * 16, 16)]
                        plsc.addupdate(acc.at[l, pl.ds(g * 16, 16)], x)
                return carry

            lax.fori_loop(0, _K // 16, gbody, 0)

        @pl.when(nch > 0)
        def _():
            pltpu.sync_copy(src_h.at[pl.ds(_off(0), _K)], s0)
            pltpu.sync_copy(dst_h.at[pl.ds(_off(0), _K)], d0)
            _gather(s0, rows0, sg0)

        @pl.when(nch > 1)
        def _():
            _sd(1, s1, d1, si1)

        def body(p, carry):
            c0 = 2 * p

            @pl.when(c0 + 1 < nch)
            def _():
                _sd_wait(c0 + 1, s1, d1, si1)
                _gather(s1, rows1, sg1)

            _gather_wait(s0, rows0, sg0)
            _compute(d0, rows0)

            @pl.when(c0 + 2 < nch)
            def _():
                _sd(c0 + 2, s0, d0, si0)

            @pl.when(c0 + 1 < nch)
            def _():
                @pl.when(c0 + 2 < nch)
                def _():
                    _sd_wait(c0 + 2, s0, d0, si0)
                    _gather(s0, rows0, sg0)

                _gather_wait(s1, rows1, sg1)
                _compute(d1, rows1)

                @pl.when(c0 + 3 < nch)
                def _():
                    _sd(c0 + 3, s1, d1, si1)

            return carry

        lax.fori_loop(0, lax.div(nch + 1, 2), body, 0)
        pltpu.sync_copy(acc.at[pl.ds(0, _W)], out_h.at[pl.ds(base, _W)])

    return k(table, gidx, dsts, bounds)


_EB = 4000


def _ti_body(a_ref, w_ref, b_ref, o_ref):
    x = a_ref[...] * w_ref[...] + b_ref[...]
    o_ref[...] = _leaky(x)


def _tc_ti(edge_attr, W3, b3):
    return pl.pallas_call(
        _ti_body,
        grid=(_E // _EB,),
        in_specs=[
            pl.BlockSpec((_EB, 1), lambda i: (i, 0)),
            pl.BlockSpec((1, _PD), lambda i: (0, 0)),
            pl.BlockSpec((1, _PD), lambda i: (0, 0)),
        ],
        out_specs=pl.BlockSpec((_EB, _PD), lambda i: (i, 0)),
        out_shape=jax.ShapeDtypeStruct((_E, _PD), jnp.float32),
    )(edge_attr, W3, b3.reshape(1, _PD))


_RB = 512


def _c_body(t_ref, af_ref, w2_ref, wb_ref, b1_ref, b2_ref, o_ref):
    o_ref[...] = (
        jnp.dot(t_ref[...], w2_ref[...], preferred_element_type=jnp.float32)
        + jnp.dot(af_ref[...], wb_ref[...], preferred_element_type=jnp.float32)
        + b1_ref[...] + b2_ref[...])


def _tc_c(aggti, af, W2, wb, b1, b2):
    nb = _NPAD // _RB
    return pl.pallas_call(
        _c_body,
        grid=(nb,),
        in_specs=[
            pl.BlockSpec((_RB, _PD), lambda i: (i, 0)),
            pl.BlockSpec((_RB, 16), lambda i: (i, 0)),
            pl.BlockSpec((_PD, _PD), lambda i: (0, 0)),
            pl.BlockSpec((16, _PD), lambda i: (0, 0)),
            pl.BlockSpec((1, _PD), lambda i: (0, 0)),
            pl.BlockSpec((1, _PD), lambda i: (0, 0)),
        ],
        out_specs=pl.BlockSpec((_RB, _PD), lambda i: (i, 0)),
        out_shape=jax.ShapeDtypeStruct((_NPAD, _PD), jnp.float32),
    )(aggti, af, W2, wb, b1.reshape(1, _PD), b2.reshape(1, _PD))


def _round_body(a_ref, w1_ref, c_ref, o_ref):
    x = jnp.dot(a_ref[...], w1_ref[...],
                preferred_element_type=jnp.float32) + c_ref[...]
    o_ref[...] = _leaky(x)


def _tc_round(agg, W1, C):
    nb = _NPAD // _RB
    return pl.pallas_call(
        _round_body,
        grid=(nb,),
        in_specs=[
            pl.BlockSpec((_RB, _PD), lambda i: (i, 0)),
            pl.BlockSpec((_PD, _PD), lambda i: (0, 0)),
            pl.BlockSpec((_RB, _PD), lambda i: (i, 0)),
        ],
        out_specs=pl.BlockSpec((_RB, _PD), lambda i: (i, 0)),
        out_shape=jax.ShapeDtypeStruct((_NPAD, _PD), jnp.float32),
    )(agg, W1, C)


def _final_body(mu_ref, bf_ref, wc_ref, bc_ref, o_ref, s_sum, s_cnt):
    i = pl.program_id(0)

    @pl.when(i == 0)
    def _():
        s_sum[...] = jnp.zeros_like(s_sum)
        s_cnt[...] = jnp.zeros_like(s_cnt)

    iot = lax.broadcasted_iota(jnp.int32, (1, _B), 1).astype(jnp.float32)
    onehot = (bf_ref[...] == iot)
    onehot = onehot.astype(jnp.float32)
    dn = (((0,), (0,)), ((), ()))
    s_sum[...] += lax.dot_general(onehot, mu_ref[...], dn,
                                  preferred_element_type=jnp.float32)
    s_cnt[...] += lax.dot_general(onehot, jnp.ones((_RB, 1), jnp.float32), dn,
                                  preferred_element_type=jnp.float32)

    @pl.when(i == pl.num_programs(0) - 1)
    def _():
        g = s_sum[...] / jnp.maximum(s_cnt[...], 1.0)
        z = jnp.dot(g, wc_ref[...], preferred_element_type=jnp.float32) + bc_ref[...]
        o_ref[...] = 1.0 / (1.0 + jnp.exp(-z))


def _tc_final(mu, bf, Wc, bc):
    nb = _NPAD // _RB
    return pl.pallas_call(
        _final_body,
        grid=(nb,),
        in_specs=[
            pl.BlockSpec((_RB, _PD), lambda i: (i, 0)),
            pl.BlockSpec((_RB, 1), lambda i: (i, 0)),
            pl.BlockSpec((_PD, 1), lambda i: (0, 0)),
            pl.BlockSpec((1, 1), lambda i: (0, 0)),
        ],
        out_specs=pl.BlockSpec((_B, 1), lambda i: (0, 0)),
        out_shape=jax.ShapeDtypeStruct((_B, 1), jnp.float32),
        scratch_shapes=[
            pltpu.VMEM((_B, _PD), jnp.float32),
            pltpu.VMEM((_B, 1), jnp.float32),
        ],
    )(mu, bf, Wc, bc.reshape(1, 1))


def kernel(x_vehicle, x_pickup, x_dropoff, edge_attr, node_mu,
           W1, b1, W2, b2, W3, b3, Wv, bv, Wp, bp, Wd, bd, Wc, bc,
           edge_index, node_types, batch):
    V = x_vehicle.shape[0]
    P = x_pickup.shape[0]
    D = x_dropoff.shape[0]
    src = edge_index[0]
    dst = edge_index[1]

    perm = jnp.argsort(dst)
    dst_sorted = dst[perm]
    padi = jnp.zeros((_K,), jnp.int32)
    dst_pad = jnp.full((_K,), _NPAD, jnp.int32)
    src_mu = jnp.concatenate([src[perm], padi])
    src_ti = jnp.concatenate([perm.astype(jnp.int32), padi])
    dst_s = jnp.concatenate([dst_sorted, dst_pad])
    edges = jnp.arange(0, _NPAD + 1, _W, dtype=jnp.int32)
    bounds = jnp.searchsorted(dst_sorted, edges).astype(jnp.int32)
    bounds = jnp.pad(bounds, (0, 64 - bounds.shape[0]))

    T = _tc_ti(edge_attr, W3, b3)
    aggti = _sc_segsum(T, src_ti, dst_s, bounds)

    af = jnp.zeros((_NPAD, 16), jnp.float32)
    af = af.at[:V, 0:2].set(x_vehicle)
    af = af.at[V:V + P, 2:5].set(x_pickup)
    af = af.at[V + P:V + P + D, 5:7].set(x_dropoff)
    af = af.at[:V, 7].set(1.0)
    af = af.at[V:V + P, 8].set(1.0)
    af = af.at[V + P:V + P + D, 9].set(1.0)
    wb = jnp.concatenate(
        [Wv, Wp, Wd, bv[None], bp[None], bd[None],
         jnp.zeros((6, _PD), jnp.float32)], axis=0)

    C = _tc_c(aggti, af, W2, wb, b1, b2)

    mu = jnp.pad(node_mu, ((0, _NPAD - _N), (0, 0)))
    for _ in range(4):
        agg = _sc_segsum(mu, src_mu, dst_s, bounds)
        mu = _tc_round(agg, W1, C)

    bf = jnp.pad(batch.astype(jnp.float32), (0, _NPAD - _N),
                 constant_values=float(_B)).reshape(_NPAD, 1)
    return _tc_final(mu, bf, Wc, bc)
